# baseline (speedup 1.0000x reference)
import jax
import jax.numpy as jnp
from jax import lax
from jax.experimental import pallas as pl
from jax.experimental.pallas import tpu as pltpu
from jax.experimental.pallas import tpu_sc as plsc

_B = 16 * 2048
_EMB = (128, 256, 128, 512)
_D = 1024
_NW = 32
_BPW = _B // _NW
_CH = (128, 128, 128, 64)


def _sc_gather_body(x0, x1, x2, x3, t0, t1, t2, t3,
                    e0, e1, e2, e3,
                    ix0, ix1, ix2, ix3,
                    a01, b01, a1, b1, a3, b3,
                    gsa, gsb, wsa, wsb):
    wid = lax.axis_index("s") * 2 + lax.axis_index("c")
    base = wid * _BPW

    for xi, ixv in ((x0, ix0), (x1, ix1), (x2, ix2), (x3, ix3)):
        pltpu.sync_copy(xi.at[pl.ds(base, _BPW)], ixv)

    def run_table(ixv, ti, ei, bufA, bufB, ch):
        npairs = _BPW // ch // 2

        def gather(j, buf, sem):
            pltpu.async_copy(ti.at[ixv.at[pl.ds(j * ch, ch)]], buf, sem)

        def wait_gather(buf, sem):
            pltpu.make_async_copy(ti.at[ixv.at[pl.ds(0, ch)]], buf, sem).wait()

        def write(j, buf, sem):
            pltpu.async_copy(buf, ei.at[pl.ds(base + j * ch, ch)], sem)

        def wait_write(buf, sem):
            pltpu.make_async_copy(buf, ei.at[pl.ds(base, ch)], sem).wait()

        gather(0, bufA, gsa)
        gather(1, bufB, gsb)

        def body(k, carry):
            j = 2 * k
            wait_gather(bufA, gsa)
            write(j, bufA, wsa)
            wait_gather(bufB, gsb)
            write(j + 1, bufB, wsb)

            @pl.when(k < npairs - 1)
            def _():
                wait_write(bufA, wsa)
                gather(j + 2, bufA, gsa)
                wait_write(bufB, wsb)
                gather(j + 3, bufB, gsb)

            return carry

        lax.fori_loop(0, npairs, body, 0)
        wait_write(bufA, wsa)
        wait_write(bufB, wsb)

    run_table(ix0, t0, e0, a01, b01, _CH[0])
    run_table(ix2, t2, e2, a01, b01, _CH[2])
    run_table(ix1, t1, e1, a1, b1, _CH[1])
    run_table(ix3, t3, e3, a3, b3, _CH[3])


def _sc_gather(x0, x1, x2, x3, t0, t1, t2, t3):
    mesh = plsc.VectorSubcoreMesh(core_axis_name="c", subcore_axis_name="s")
    k = pl.kernel(
        _sc_gather_body,
        out_type=[
            jax.ShapeDtypeStruct((_B, _EMB[0]), jnp.float32),
            jax.ShapeDtypeStruct((_B, _EMB[1] // 2), jnp.int32),
            jax.ShapeDtypeStruct((_B, _EMB[2]), jnp.float32),
            jax.ShapeDtypeStruct((_B, _EMB[3] // 2), jnp.int32),
        ],
        mesh=mesh,
        scratch_types=[
            pltpu.VMEM((_BPW,), jnp.int32),
            pltpu.VMEM((_BPW,), jnp.int32),
            pltpu.VMEM((_BPW,), jnp.int32),
            pltpu.VMEM((_BPW,), jnp.int32),
            pltpu.VMEM((_CH[0], _EMB[0]), jnp.float32),
            pltpu.VMEM((_CH[0], _EMB[0]), jnp.float32),
            pltpu.VMEM((_CH[1], _EMB[1] // 2), jnp.int32),
            pltpu.VMEM((_CH[1], _EMB[1] // 2), jnp.int32),
            pltpu.VMEM((_CH[3], _EMB[3] // 2), jnp.int32),
            pltpu.VMEM((_CH[3], _EMB[3] // 2), jnp.int32),
            pltpu.SemaphoreType.DMA,
            pltpu.SemaphoreType.DMA,
            pltpu.SemaphoreType.DMA,
            pltpu.SemaphoreType.DMA,
        ],
    )
    return k(x0, x1, x2, x3, t0, t1, t2, t3)


def _mm_body(e0, e1, e2, e3, w, bias, out):
    acc = jnp.dot(e0[...].astype(jnp.bfloat16), w[0:128, :],
                  preferred_element_type=jnp.float32)
    acc = acc + jnp.dot(e1[...], w[128:384, :], preferred_element_type=jnp.float32)
    acc = acc + jnp.dot(e2[...].astype(jnp.bfloat16), w[384:512, :],
                  preferred_element_type=jnp.float32)
    acc = acc + jnp.dot(e3[...], w[512:1024, :], preferred_element_type=jnp.float32)
    out[...] = acc + bias[...]


def _tc_matmul(e0, e1, e2, e3, W, b):
    bm = 2048
    grid = (_B // bm,)
    return pl.pallas_call(
        _mm_body,
        grid=grid,
        in_specs=[
            pl.BlockSpec((bm, _EMB[0]), lambda i: (i, 0)),
            pl.BlockSpec((bm, _EMB[1]), lambda i: (i, 0)),
            pl.BlockSpec((bm, _EMB[2]), lambda i: (i, 0)),
            pl.BlockSpec((bm, _EMB[3]), lambda i: (i, 0)),
            pl.BlockSpec((sum(_EMB), _D), lambda i: (0, 0)),
            pl.BlockSpec((1, _D), lambda i: (0, 0)),
        ],
        out_specs=pl.BlockSpec((bm, _D), lambda i: (i, 0)),
        out_shape=jax.ShapeDtypeStruct((_B, _D), jnp.float32),
        compiler_params=pltpu.CompilerParams(
            dimension_semantics=("arbitrary",),
        ),
    )(e0, e1, e2, e3, W, b.reshape(1, _D))


def _pack_table(t):
    n, e = t.shape
    tb = t.astype(jnp.bfloat16).reshape(n, e // 2, 2)
    return jax.lax.bitcast_convert_type(tb, jnp.int32)


def _unpack_e(e_i32):
    n = e_i32.shape[0]
    eb = jax.lax.bitcast_convert_type(e_i32, jnp.bfloat16)
    return eb.reshape(n, -1)


def kernel(x, t0, t1, t2, t3, W, b):
    lead = x.shape[:-1]
    xr = x.reshape(-1, 4).astype(jnp.int32)
    x0, x1, x2, x3 = xr[:, 0], xr[:, 1], xr[:, 2], xr[:, 3]
    e0, e1, e2, e3 = _sc_gather(
        x0, x1, x2, x3,
        t0, _pack_table(t1), t2, _pack_table(t3))
    out = _tc_matmul(e0, _unpack_e(e1), e2, _unpack_e(e3),
                     W.astype(jnp.bfloat16), b)
    return out.reshape(*lead, _D)


# speedup vs baseline: 3.5850x; 3.5850x over previous
import jax
import jax.numpy as jnp
from jax import lax
from jax.experimental import pallas as pl
from jax.experimental.pallas import tpu as pltpu
from jax.experimental.pallas import tpu_sc as plsc

_B = 16 * 2048
_EMB = (128, 256, 128, 512)
_D = 1024
_NW = 32
_BPW = _B // _NW
_CH = (64, 64, 64, 64)


def _sc_gather_body(x0, x1, x2, x3, t0, t1, t2, t3,
                    e0, e1, e2, e3,
                    ix0, ix1, ix2, ix3,
                    a01, b01, a1, b1, a3, b3,
                    gsa, gsb, wsa, wsb):
    wid = lax.axis_index("s") * 2 + lax.axis_index("c")
    base = wid * _BPW

    for xi, ixv in ((x0, ix0), (x1, ix1), (x2, ix2), (x3, ix3)):
        pltpu.sync_copy(xi.at[pl.ds(base, _BPW)], ixv)

    def run_table(ixv, ti, ei, bufA, bufB, ch):
        npairs = _BPW // ch // 2

        def gather(j, buf, sem):
            pltpu.async_copy(ti.at[ixv.at[pl.ds(j * ch, ch)]], buf, sem)

        def wait_gather(buf, sem):
            pltpu.make_async_copy(ti.at[ixv.at[pl.ds(0, ch)]], buf, sem).wait()

        def write(j, buf, sem):
            pltpu.async_copy(buf, ei.at[pl.ds(base + j * ch, ch)], sem)

        def wait_write(buf, sem):
            pltpu.make_async_copy(buf, ei.at[pl.ds(base, ch)], sem).wait()

        gather(0, bufA, gsa)
        gather(1, bufB, gsb)

        def body(k, carry):
            j = 2 * k
            wait_gather(bufA, gsa)
            write(j, bufA, wsa)
            wait_gather(bufB, gsb)
            write(j + 1, bufB, wsb)

            @pl.when(k < npairs - 1)
            def _():
                wait_write(bufA, wsa)
                gather(j + 2, bufA, gsa)
                wait_write(bufB, wsb)
                gather(j + 3, bufB, gsb)

            return carry

        lax.fori_loop(0, npairs, body, 0)
        wait_write(bufA, wsa)
        wait_write(bufB, wsb)

    run_table(ix0, t0, e0, a01, b01, _CH[0])
    run_table(ix2, t2, e2, a01, b01, _CH[2])
    run_table(ix1, t1, e1, a1, b1, _CH[1])
    run_table(ix3, t3, e3, a3, b3, _CH[3])


def _sc_gather(x0, x1, x2, x3, t0, t1, t2, t3):
    mesh = plsc.VectorSubcoreMesh(core_axis_name="c", subcore_axis_name="s")
    k = pl.kernel(
        _sc_gather_body,
        out_type=[jax.ShapeDtypeStruct((_B, e), jnp.float32) for e in _EMB],
        mesh=mesh,
        scratch_types=[
            pltpu.VMEM((_BPW,), jnp.int32),
            pltpu.VMEM((_BPW,), jnp.int32),
            pltpu.VMEM((_BPW,), jnp.int32),
            pltpu.VMEM((_BPW,), jnp.int32),
            pltpu.VMEM((_CH[0], _EMB[0]), jnp.float32),
            pltpu.VMEM((_CH[0], _EMB[0]), jnp.float32),
            pltpu.VMEM((_CH[1], _EMB[1]), jnp.float32),
            pltpu.VMEM((_CH[1], _EMB[1]), jnp.float32),
            pltpu.VMEM((_CH[3], _EMB[3]), jnp.float32),
            pltpu.VMEM((_CH[3], _EMB[3]), jnp.float32),
            pltpu.SemaphoreType.DMA,
            pltpu.SemaphoreType.DMA,
            pltpu.SemaphoreType.DMA,
            pltpu.SemaphoreType.DMA,
        ],
    )
    return k(x0, x1, x2, x3, t0, t1, t2, t3)


def _mm_body(e0, e1, e2, e3, w, bias, out):
    acc = jnp.dot(e0[...], w[0:128, :], preferred_element_type=jnp.float32)
    acc = acc + jnp.dot(e1[...], w[128:384, :], preferred_element_type=jnp.float32)
    acc = acc + jnp.dot(e2[...], w[384:512, :], preferred_element_type=jnp.float32)
    acc = acc + jnp.dot(e3[...], w[512:1024, :], preferred_element_type=jnp.float32)
    out[...] = acc + bias[...]


def _tc_matmul(e0, e1, e2, e3, W, b):
    bm = 2048
    grid = (_B // bm,)
    return pl.pallas_call(
        _mm_body,
        grid=grid,
        in_specs=[
            pl.BlockSpec((bm, _EMB[0]), lambda i: (i, 0)),
            pl.BlockSpec((bm, _EMB[1]), lambda i: (i, 0)),
            pl.BlockSpec((bm, _EMB[2]), lambda i: (i, 0)),
            pl.BlockSpec((bm, _EMB[3]), lambda i: (i, 0)),
            pl.BlockSpec((sum(_EMB), _D), lambda i: (0, 0)),
            pl.BlockSpec((1, _D), lambda i: (0, 0)),
        ],
        out_specs=pl.BlockSpec((bm, _D), lambda i: (i, 0)),
        out_shape=jax.ShapeDtypeStruct((_B, _D), jnp.float32),
        compiler_params=pltpu.CompilerParams(
            dimension_semantics=("arbitrary",),
        ),
    )(e0, e1, e2, e3, W, b.reshape(1, _D))


def kernel(x, t0, t1, t2, t3, W, b):
    lead = x.shape[:-1]
    xr = x.reshape(-1, 4).astype(jnp.int32)
    x0, x1, x2, x3 = xr[:, 0], xr[:, 1], xr[:, 2], xr[:, 3]
    e0, e1, e2, e3 = _sc_gather(x0, x1, x2, x3, t0, t1, t2, t3)
    out = _tc_matmul(e0, e1, e2, e3, W, b)
    return out.reshape(*lead, _D)
